# Initial kernel scaffold; baseline (speedup 1.0000x reference)
import functools
import math

import jax
import jax.numpy as jnp
from jax import lax
from jax.experimental import pallas as pl
from jax.experimental.pallas import tpu as pltpu
from jax.experimental.pallas import tpu_sc as plsc

D = 64
L = 16
NC, NS = 2, 16
NW = NC * NS
W = 256
GCH = 128
SCALE = math.sqrt(float(D))


def _wait(src, dst, sem):
    pltpu.make_async_copy(src, dst, sem).wait()


def _pe_add_kernel(N, rows_per_w, nblk):
    mesh = plsc.VectorSubcoreMesh(core_axis_name="c", subcore_axis_name="s")

    @functools.partial(
        pl.kernel,
        out_type=jax.ShapeDtypeStruct((N, D), jnp.float32),
        mesh=mesh,
        scratch_types=[
            pltpu.VMEM((2, W // GCH, GCH), jnp.int32),
            pltpu.VMEM((2, W, D), jnp.float32),
            pltpu.VMEM((2, W, D), jnp.float32),
            pltpu.VMEM((2, W, D), jnp.float32),
            pltpu.SemaphoreType.DMA,
            pltpu.SemaphoreType.DMA,
            pltpu.SemaphoreType.DMA,
            pltpu.SemaphoreType.DMA,
            pltpu.SemaphoreType.DMA,
            pltpu.SemaphoreType.DMA,
            pltpu.SemaphoreType.DMA,
            pltpu.SemaphoreType.DMA,
        ],
    )
    def k(x_hbm, s_hbm, pe_hbm, o_hbm, idx_v, x_v, e_v, o_v,
          sem_i0, sem_i1, sem_x0, sem_x1, sem_g0, sem_g1, sem_o0, sem_o1):
        sem_i = (sem_i0, sem_i1)
        sem_x = (sem_x0, sem_x1)
        sem_g = (sem_g0, sem_g1)
        sem_o = (sem_o0, sem_o1)
        wid = lax.axis_index("s") * NC + lax.axis_index("c")
        base = wid * rows_per_w
        cbase = wid * (rows_per_w // GCH)

        def issue_in(b, p):
            pltpu.async_copy(
                s_hbm.at[pl.ds(cbase + b * (W // GCH), W // GCH)],
                idx_v.at[p], sem_i[p])
            pltpu.async_copy(
                x_hbm.at[pl.ds(base + b * W, W)], x_v.at[p], sem_x[p])

        def issue_gather(p):
            for j in range(W // GCH):
                pltpu.async_copy(
                    pe_hbm.at[idx_v.at[p].at[j]],
                    e_v.at[p].at[pl.ds(j * GCH, GCH)], sem_g[p])

        def wait_gather(p):
            for j in range(W // GCH):
                _wait(pe_hbm.at[idx_v.at[p].at[j]],
                      e_v.at[p].at[pl.ds(j * GCH, GCH)], sem_g[p])

        issue_in(0, 0)
        _wait(s_hbm.at[pl.ds(cbase, W // GCH)], idx_v.at[0], sem_i[0])
        issue_gather(0)
        issue_in(1, 1)

        @pl.loop(0, nblk, step=2)
        def _(bb):
            for p in range(2):
                b = bb + p
                _wait(x_hbm.at[pl.ds(base + b * W, W)], x_v.at[p], sem_x[p])
                wait_gather(p)

                @pl.when(b + 1 < nblk)
                def _():
                    q = 1 - p
                    _wait(s_hbm.at[pl.ds(cbase + (b + 1) * (W // GCH),
                                         W // GCH)], idx_v.at[q], sem_i[q])
                    issue_gather(q)

                @pl.when(b >= 2)
                def _():
                    _wait(o_v.at[p],
                          o_hbm.at[pl.ds(base + (b - 2) * W, W)], sem_o[p])

                xb, eb, ob = x_v.at[p], e_v.at[p], o_v.at[p]

                @pl.loop(0, W)
                def _(r):
                    for c in range(D // L):
                        sl = pl.ds(c * L, L)
                        ob[r, sl] = eb[r, sl] + xb[r, sl] * SCALE

                pltpu.async_copy(o_v.at[p],
                                 o_hbm.at[pl.ds(base + b * W, W)], sem_o[p])

                @pl.when(b + 2 < nblk)
                def _():
                    issue_in(b + 2, p)

        for b in (nblk - 2, nblk - 1):
            p = b % 2
            _wait(o_v.at[p], o_hbm.at[pl.ds(base + b * W, W)], sem_o[p])

    return k


def kernel(x, steps, len, pe):
    B, S, d = x.shape
    N = B * S
    rows_per_w = N // NW
    nblk = rows_per_w // W
    xf = x.reshape(N, D)
    sf = steps.reshape(N // GCH, GCH)
    pef = pe.reshape(pe.shape[0], D)
    out = _pe_add_kernel(N, rows_per_w, nblk)(xf, sf, pef)
    return out.reshape(B, S, D)


# speedup vs baseline: 2.8811x; 2.8811x over previous
import functools
import math

import jax
import jax.numpy as jnp
from jax import lax
from jax.experimental import pallas as pl
from jax.experimental.pallas import tpu as pltpu
from jax.experimental.pallas import tpu_sc as plsc

D = 64
L = 16
NC, NS = 2, 16
NW = NC * NS
W = 256
GCH = 128
SCALE = math.sqrt(float(D))


def _wait(src, dst, sem):
    pltpu.make_async_copy(src, dst, sem).wait()


def _pe_add_kernel(N, rows_per_w, nblk):
    mesh = plsc.VectorSubcoreMesh(core_axis_name="c", subcore_axis_name="s")

    @functools.partial(
        pl.kernel,
        out_type=jax.ShapeDtypeStruct((N, D), jnp.float32),
        mesh=mesh,
        compiler_params=pltpu.CompilerParams(use_tc_tiling_on_sc=False),
        scratch_types=[
            pltpu.VMEM((2, W // GCH, GCH), jnp.int32),
            pltpu.VMEM((2, W, D), jnp.float32),
            pltpu.VMEM((2, W, D), jnp.float32),
            pltpu.VMEM((2, W, D), jnp.float32),
            pltpu.SemaphoreType.DMA,
            pltpu.SemaphoreType.DMA,
            pltpu.SemaphoreType.DMA,
            pltpu.SemaphoreType.DMA,
            pltpu.SemaphoreType.DMA,
            pltpu.SemaphoreType.DMA,
            pltpu.SemaphoreType.DMA,
            pltpu.SemaphoreType.DMA,
        ],
    )
    def k(x_hbm, s_hbm, pe_hbm, o_hbm, idx_v, x_v, e_v, o_v,
          sem_i0, sem_i1, sem_x0, sem_x1, sem_g0, sem_g1, sem_o0, sem_o1):
        sem_i = (sem_i0, sem_i1)
        sem_x = (sem_x0, sem_x1)
        sem_g = (sem_g0, sem_g1)
        sem_o = (sem_o0, sem_o1)
        wid = lax.axis_index("s") * NC + lax.axis_index("c")
        base = wid * rows_per_w
        cbase = wid * (rows_per_w // GCH)

        def issue_in(b, p):
            pltpu.async_copy(
                s_hbm.at[pl.ds(cbase + b * (W // GCH), W // GCH)],
                idx_v.at[p], sem_i[p])
            pltpu.async_copy(
                x_hbm.at[pl.ds(base + b * W, W)], x_v.at[p], sem_x[p])

        def issue_gather(p):
            for j in range(W // GCH):
                pltpu.async_copy(
                    pe_hbm.at[idx_v.at[p].at[j]],
                    e_v.at[p].at[pl.ds(j * GCH, GCH)], sem_g[p])

        def wait_gather(p):
            for j in range(W // GCH):
                _wait(pe_hbm.at[idx_v.at[p].at[j]],
                      e_v.at[p].at[pl.ds(j * GCH, GCH)], sem_g[p])

        issue_in(0, 0)
        _wait(s_hbm.at[pl.ds(cbase, W // GCH)], idx_v.at[0], sem_i[0])
        issue_gather(0)
        issue_in(1, 1)

        @pl.loop(0, nblk, step=2)
        def _(bb):
            for p in range(2):
                b = bb + p
                _wait(x_hbm.at[pl.ds(base + b * W, W)], x_v.at[p], sem_x[p])
                wait_gather(p)

                @pl.when(b + 1 < nblk)
                def _():
                    q = 1 - p
                    _wait(s_hbm.at[pl.ds(cbase + (b + 1) * (W // GCH),
                                         W // GCH)], idx_v.at[q], sem_i[q])
                    issue_gather(q)

                @pl.when(b >= 2)
                def _():
                    _wait(o_v.at[p],
                          o_hbm.at[pl.ds(base + (b - 2) * W, W)], sem_o[p])

                xb, eb, ob = x_v.at[p], e_v.at[p], o_v.at[p]

                @pl.loop(0, W)
                def _(r):
                    for c in range(D // L):
                        sl = pl.ds(c * L, L)
                        ob[r, sl] = eb[r, sl] + xb[r, sl] * SCALE

                pltpu.async_copy(o_v.at[p],
                                 o_hbm.at[pl.ds(base + b * W, W)], sem_o[p])

                @pl.when(b + 2 < nblk)
                def _():
                    issue_in(b + 2, p)

        for b in (nblk - 2, nblk - 1):
            p = b % 2
            _wait(o_v.at[p], o_hbm.at[pl.ds(base + b * W, W)], sem_o[p])

    return k


def kernel(x, steps, len, pe):
    B, S, d = x.shape
    N = B * S
    rows_per_w = N // NW
    nblk = rows_per_w // W
    xf = x.reshape(N, D)
    sf = steps.reshape(N // GCH, GCH)
    pef = pe.reshape(pe.shape[0], D)
    out = _pe_add_kernel(N, rows_per_w, nblk)(xf, sf, pef)
    return out.reshape(B, S, D)


# speedup vs baseline: 4.3954x; 1.5256x over previous
import functools
import math

import jax
import jax.numpy as jnp
from jax import lax
from jax.experimental import pallas as pl
from jax.experimental.pallas import tpu as pltpu
from jax.experimental.pallas import tpu_sc as plsc

D = 64
DP = 128
L = 16
NC, NS = 2, 16
NW = NC * NS
W = 128
GCH = 128
SCALE = math.sqrt(float(D))
RUNROLL = 4


def _wait(src, dst, sem):
    pltpu.make_async_copy(src, dst, sem).wait()


def _pe_add_kernel(N, rows_per_w, nblk):
    mesh = plsc.VectorSubcoreMesh(core_axis_name="c", subcore_axis_name="s")

    @functools.partial(
        pl.kernel,
        out_type=jax.ShapeDtypeStruct((N, D), jnp.float32),
        mesh=mesh,
        scratch_types=[
            pltpu.VMEM((2, W // GCH, GCH), jnp.int32),
            pltpu.VMEM((2, W, D), jnp.float32),
            pltpu.VMEM((2, W, DP), jnp.float32),
            pltpu.VMEM((2, W, D), jnp.float32),
            pltpu.SemaphoreType.DMA,
            pltpu.SemaphoreType.DMA,
            pltpu.SemaphoreType.DMA,
            pltpu.SemaphoreType.DMA,
            pltpu.SemaphoreType.DMA,
            pltpu.SemaphoreType.DMA,
            pltpu.SemaphoreType.DMA,
            pltpu.SemaphoreType.DMA,
        ],
    )
    def k(x_hbm, s_hbm, pe_hbm, o_hbm, idx_v, x_v, e_v, o_v,
          sem_i0, sem_i1, sem_x0, sem_x1, sem_g0, sem_g1, sem_o0, sem_o1):
        sem_i = (sem_i0, sem_i1)
        sem_x = (sem_x0, sem_x1)
        sem_g = (sem_g0, sem_g1)
        sem_o = (sem_o0, sem_o1)
        wid = lax.axis_index("s") * NC + lax.axis_index("c")
        base = wid * rows_per_w
        cbase = wid * (rows_per_w // GCH)

        def issue_in(b, p):
            pltpu.async_copy(
                s_hbm.at[pl.ds(cbase + b * (W // GCH), W // GCH)],
                idx_v.at[p], sem_i[p])
            pltpu.async_copy(
                x_hbm.at[pl.ds(base + b * W, W)], x_v.at[p], sem_x[p])

        def issue_gather(p):
            for j in range(W // GCH):
                pltpu.async_copy(
                    pe_hbm.at[idx_v.at[p].at[j]],
                    e_v.at[p].at[pl.ds(j * GCH, GCH)], sem_g[p])

        def wait_gather(p):
            for j in range(W // GCH):
                _wait(pe_hbm.at[idx_v.at[p].at[j]],
                      e_v.at[p].at[pl.ds(j * GCH, GCH)], sem_g[p])

        issue_in(0, 0)
        _wait(s_hbm.at[pl.ds(cbase, W // GCH)], idx_v.at[0], sem_i[0])
        issue_gather(0)
        issue_in(1, 1)

        @pl.loop(0, nblk, step=2)
        def _(bb):
            for p in range(2):
                b = bb + p
                _wait(x_hbm.at[pl.ds(base + b * W, W)], x_v.at[p], sem_x[p])
                wait_gather(p)

                @pl.when(b + 1 < nblk)
                def _():
                    q = 1 - p
                    _wait(s_hbm.at[pl.ds(cbase + (b + 1) * (W // GCH),
                                         W // GCH)], idx_v.at[q], sem_i[q])
                    issue_gather(q)

                @pl.when(b >= 2)
                def _():
                    _wait(o_v.at[p],
                          o_hbm.at[pl.ds(base + (b - 2) * W, W)], sem_o[p])

                xb, eb, ob = x_v.at[p], e_v.at[p], o_v.at[p]

                @pl.loop(0, W, step=RUNROLL)
                def _(r0):
                    for dr in range(RUNROLL):
                        r = r0 + dr
                        for c in range(D // L):
                            sl = pl.ds(c * L, L)
                            ob[r, sl] = eb[r, sl] + xb[r, sl] * SCALE

                pltpu.async_copy(o_v.at[p],
                                 o_hbm.at[pl.ds(base + b * W, W)], sem_o[p])

                @pl.when(b + 2 < nblk)
                def _():
                    issue_in(b + 2, p)

        for b in (nblk - 2, nblk - 1):
            p = b % 2
            _wait(o_v.at[p], o_hbm.at[pl.ds(base + b * W, W)], sem_o[p])

    return k


def kernel(x, steps, len, pe):
    B, S, d = x.shape
    N = B * S
    rows_per_w = N // NW
    nblk = rows_per_w // W
    xf = x.reshape(N, D)
    sf = steps.reshape(N // GCH, GCH)
    pef = jnp.pad(pe.reshape(pe.shape[0], D), ((0, 0), (0, DP - D)))
    out = _pe_add_kernel(N, rows_per_w, nblk)(xf, sf, pef)
    return out.reshape(B, S, D)


# speedup vs baseline: 5.1979x; 1.1826x over previous
import dataclasses
import functools
import math

import jax
import jax.numpy as jnp
from jax import lax
from jax.experimental import pallas as pl
from jax.experimental.pallas import tpu as pltpu
from jax.experimental.pallas import tpu_sc as plsc

D = 64
L = 16
NC, NS = 2, 16
NW = NC * NS
SBLK = 8
DG = 8
BW = 1024
VPAD = 5120
SCALE = math.sqrt(float(D))


def _wait(src, dst, sem):
    pltpu.make_async_copy(src, dst, sem).wait()


def _pe_add_kernel(S, B):
    BQ = B // 4
    NBW = BQ // BW
    NU = (S // SBLK) * NBW * SBLK
    mesh = plsc.VectorSubcoreMesh(core_axis_name="c", subcore_axis_name="s")
    cp = pltpu.CompilerParams()
    if "needs_layout_passes" in pltpu.CompilerParams.__dataclass_fields__:
        cp = dataclasses.replace(cp, needs_layout_passes=False)

    @functools.partial(
        pl.kernel,
        out_type=jax.ShapeDtypeStruct((S * D, B), jnp.float32),
        mesh=mesh,
        compiler_params=cp,
        scratch_types=[
            pltpu.VMEM((DG * VPAD,), jnp.float32),
            pltpu.VMEM((DG, BW), jnp.float32),
            pltpu.VMEM((DG, BW), jnp.float32),
            pltpu.VMEM((DG, BW), jnp.float32),
            pltpu.VMEM((DG, BW), jnp.float32),
            pltpu.VMEM((SBLK, BW), jnp.int32),
            pltpu.SemaphoreType.DMA,
            pltpu.SemaphoreType.DMA,
            pltpu.SemaphoreType.DMA,
            pltpu.SemaphoreType.DMA,
            pltpu.SemaphoreType.DMA,
            pltpu.SemaphoreType.DMA,
        ],
    )
    def k(x_hbm, s_hbm, tab_hbm, o_hbm, tab_v, x_v0, x_v1, o_v0, o_v1, st_v,
          sem_t, sem_x0, sem_x1, sem_o0, sem_o1, sem_s):
        x_v = (x_v0, x_v1)
        o_v = (o_v0, o_v1)
        sem_x = (sem_x0, sem_x1)
        sem_o = (sem_o0, sem_o1)
        wid = lax.axis_index("s") * NC + lax.axis_index("c")
        g = lax.rem(wid, 8)
        q = wid // 8
        d0 = g * DG
        bq0 = q * BQ

        def row_of(u):
            return ((u >> 5) * SBLK + (u & 7)) * D + d0

        def b_of(u):
            return bq0 + ((u >> 3) & (NBW - 1)) * BW

        def x_win(u):
            return x_hbm.at[pl.ds(row_of(u), DG), pl.ds(b_of(u), BW)]

        def o_win(u):
            return o_hbm.at[pl.ds(row_of(u), DG), pl.ds(b_of(u), BW)]

        def st_win(u):
            return s_hbm.at[pl.ds((u >> 5) * SBLK, SBLK), pl.ds(b_of(u), BW)]

        pltpu.async_copy(tab_hbm.at[pl.ds(d0 * VPAD, DG * VPAD)],
                         tab_v, sem_t).wait()

        pltpu.async_copy(st_win(0), st_v, sem_s).wait()
        pltpu.async_copy(x_win(0), x_v[0], sem_x[0])
        pltpu.async_copy(x_win(1), x_v[1], sem_x[1])

        @pl.loop(0, NU, step=2)
        def _(u0):
            for p in range(2):
                u = u0 + p
                s8 = u & 7

                if p == 0:
                    @pl.when((s8 == 0) & (u > 0))
                    def _():
                        pltpu.async_copy(st_win(u), st_v, sem_s).wait()

                _wait(x_win(u), x_v[p], sem_x[p])

                @pl.when(u >= 2)
                def _():
                    _wait(o_v[p], o_win(u - 2), sem_o[p])

                xb, ob = x_v[p], o_v[p]

                @pl.loop(0, BW // L)
                def _(kk):
                    sl = pl.ds(kk * L, L)
                    idx = st_v[s8, sl]
                    for dl in range(DG):
                        pe_val = plsc.load_gather(tab_v, [idx + dl * VPAD])
                        ob[dl, sl] = xb[dl, sl] * SCALE + pe_val

                pltpu.async_copy(ob, o_win(u), sem_o[p])

                @pl.when(u + 2 < NU)
                def _():
                    pltpu.async_copy(x_win(u + 2), x_v[p], sem_x[p])

        for u in (NU - 2, NU - 1):
            _wait(o_v[u % 2], o_win(u), sem_o[u % 2])

    return k


def kernel(x, steps, len, pe):
    B, S, d = x.shape
    xt = jnp.transpose(x, (1, 2, 0)).reshape(S * D, B)
    st = jnp.transpose(steps, (1, 0))
    tab = jnp.pad(jnp.transpose(pe[:, 0, :]), ((0, 0), (0, VPAD - pe.shape[0])))
    tab = tab.reshape(-1)
    ot = _pe_add_kernel(S, B)(xt, st, tab)
    return jnp.transpose(ot.reshape(S, D, B), (2, 0, 1))


# speedup vs baseline: 12.0288x; 2.3142x over previous
import dataclasses
import functools
import math

import jax
import jax.numpy as jnp
from jax import lax
from jax.experimental import pallas as pl
from jax.experimental.pallas import tpu as pltpu
from jax.experimental.pallas import tpu_sc as plsc

D = 64
L = 16
NC, NS = 2, 16
NW = NC * NS
SBLK = 8
DG = 8
BW = 1024
VPAD = 5120
SCALE = math.sqrt(float(D))


def _wait(src, dst, sem):
    pltpu.make_async_copy(src, dst, sem).wait()


def _pe_add_kernel(S, B):
    BQ = B // 4
    NBW = BQ // BW
    NU = (S // SBLK) * NBW * SBLK
    mesh = plsc.VectorSubcoreMesh(core_axis_name="c", subcore_axis_name="s")
    cp = pltpu.CompilerParams()
    if "needs_layout_passes" in pltpu.CompilerParams.__dataclass_fields__:
        cp = dataclasses.replace(cp, needs_layout_passes=False)

    @functools.partial(
        pl.kernel,
        out_type=jax.ShapeDtypeStruct((S * D, B), jnp.float32),
        mesh=mesh,
        compiler_params=cp,
        scratch_types=[
            pltpu.VMEM((DG * VPAD,), jnp.float32),
            pltpu.VMEM((DG, BW), jnp.float32),
            pltpu.VMEM((DG, BW), jnp.float32),
            pltpu.VMEM((DG, BW), jnp.float32),
            pltpu.VMEM((DG, BW), jnp.float32),
            pltpu.VMEM((SBLK, BW), jnp.int32),
            pltpu.SemaphoreType.DMA,
            pltpu.SemaphoreType.DMA,
            pltpu.SemaphoreType.DMA,
            pltpu.SemaphoreType.DMA,
            pltpu.SemaphoreType.DMA,
            pltpu.SemaphoreType.DMA,
        ],
    )
    def k(x_hbm, s_hbm, tab_hbm, o_hbm, tab_v, x_v0, x_v1, o_v0, o_v1, st_v,
          sem_t, sem_x0, sem_x1, sem_o0, sem_o1, sem_s):
        x_v = (x_v0, x_v1)
        o_v = (o_v0, o_v1)
        sem_x = (sem_x0, sem_x1)
        sem_o = (sem_o0, sem_o1)
        wid = lax.axis_index("s") * NC + lax.axis_index("c")
        g = lax.rem(wid, 8)
        q = wid // 8
        d0 = g * DG
        bq0 = q * BQ

        def row_of(u):
            return ((u >> 5) * SBLK + (u & 7)) * D + d0

        def b_of(u):
            return bq0 + ((u >> 3) & (NBW - 1)) * BW

        def x_win(u):
            return x_hbm.at[pl.ds(row_of(u), DG), pl.ds(b_of(u), BW)]

        def o_win(u):
            return o_hbm.at[pl.ds(row_of(u), DG), pl.ds(b_of(u), BW)]

        def st_win(u):
            return s_hbm.at[pl.ds((u >> 5) * SBLK, SBLK), pl.ds(b_of(u), BW)]

        pltpu.async_copy(tab_hbm.at[pl.ds(d0 * VPAD, DG * VPAD)],
                         tab_v, sem_t).wait()

        pltpu.async_copy(st_win(0), st_v, sem_s).wait()
        pltpu.async_copy(x_win(0), x_v[0], sem_x[0])
        pltpu.async_copy(x_win(1), x_v[1], sem_x[1])

        @pl.loop(0, NU, step=2)
        def _(u0):
            for p in range(2):
                u = u0 + p
                s8 = u & 7

                if p == 0:
                    @pl.when((s8 == 0) & (u > 0))
                    def _():
                        pltpu.async_copy(st_win(u), st_v, sem_s).wait()

                _wait(x_win(u), x_v[p], sem_x[p])

                @pl.when(u >= 2)
                def _():
                    _wait(o_v[p], o_win(u - 2), sem_o[p])

                xb, ob = x_v[p], o_v[p]

                @pl.loop(0, BW // L, step=2)
                def _(kk):
                    vals = []
                    for kb in range(2):
                        sl = pl.ds((kk + kb) * L, L)
                        idx = st_v[s8, sl]
                        for dl in range(DG):
                            pe_val = plsc.load_gather(
                                tab_v, [idx + dl * VPAD])
                            vals.append(xb[dl, sl] * SCALE + pe_val)
                    for kb in range(2):
                        sl = pl.ds((kk + kb) * L, L)
                        for dl in range(DG):
                            ob[dl, sl] = vals[kb * DG + dl]

                pltpu.async_copy(ob, o_win(u), sem_o[p])

                @pl.when(u + 2 < NU)
                def _():
                    pltpu.async_copy(x_win(u + 2), x_v[p], sem_x[p])

        for u in (NU - 2, NU - 1):
            _wait(o_v[u % 2], o_win(u), sem_o[u % 2])

    return k


def kernel(x, steps, len, pe):
    B, S, d = x.shape
    xt = jnp.transpose(x, (1, 2, 0)).reshape(S * D, B)
    st = jnp.transpose(steps, (1, 0))
    tab = jnp.pad(jnp.transpose(pe[:, 0, :]), ((0, 0), (0, VPAD - pe.shape[0])))
    tab = tab.reshape(-1)
    ot = _pe_add_kernel(S, B)(xt, st, tab)
    return jnp.transpose(ot.reshape(S, D, B), (2, 0, 1))
